# Initial kernel scaffold; baseline (speedup 1.0000x reference)
import jax
import jax.numpy as jnp
from jax.experimental import pallas as pl


def kernel(x, edge_index, bit_assign, W, b):
    raise NotImplementedError("write your pallas kernel here")


# speedup vs baseline: 24.5608x; 24.5608x over previous
import functools

import jax
import jax.numpy as jnp
from jax import lax
from jax.experimental import pallas as pl
from jax.experimental.pallas import tpu as pltpu
from jax.experimental.pallas import tpu_sc as plsc

N = 10000
E = 320000
D = 128
NC, NS, L = 2, 16, 16
NW = NC * NS
EPW = E // NW
CH = 80
NCHUNK = EPW // CH
RPT = 640
ZR = 80

def _sc_degree_body(dstr_hbm, out_hbm, dst_v, hist_v):
    c = lax.axis_index("c")
    s = lax.axis_index("s")
    wid = c * NS + s
    pltpu.sync_copy(dstr_hbm.at[wid], dst_v)
    zeros = jnp.zeros((L,), jnp.float32)

    def zbody(i, _):
        hist_v[pl.ds(i * L, L)] = zeros
        return 0

    lax.fori_loop(0, N // L, zbody, 0)
    ones = jnp.ones((L,), jnp.float32)

    def hbody(j, _):
        for k in range(CH // L):
            idx = dst_v[j, pl.ds(k * L, L)]
            plsc.addupdate_scatter(hist_v, [idx], ones)
        return 0

    lax.fori_loop(0, NCHUNK, hbody, 0)
    pltpu.sync_copy(hist_v, out_hbm.at[pl.ds(wid * N, N)])


def _sc_scatter_body(g_hbm, srcr_hbm, dstr_hbm, out_hbm,
                     src_v, dst_v, buf, acc_sh, gsem):
    c = lax.axis_index("c")
    s = lax.axis_index("s")
    wid = c * NS + s
    pltpu.sync_copy(srcr_hbm.at[wid], src_v)
    pltpu.sync_copy(dstr_hbm.at[wid], dst_v)

    zeros = jnp.zeros((L,), jnp.float32)

    def zbody(i, _):
        for k in range(D // L):
            buf[i, pl.ds(k * L, L)] = zeros
        return 0

    lax.fori_loop(0, ZR, zbody, 0)
    base = s * RPT
    for t in range(5):
        pltpu.sync_copy(buf, acc_sh.at[pl.ds(base + t * ZR, ZR)])

    @pl.when(s < NS - 1)
    def _():
        for t in range(5, 8):
            pltpu.sync_copy(buf, acc_sh.at[pl.ds(base + t * ZR, ZR)])

    plsc.subcore_barrier()

    def body(j, _):
        pltpu.async_copy(g_hbm.at[src_v.at[j]], buf, gsem).wait()
        pltpu.sync_copy(buf, acc_sh.at[dst_v.at[j]], add=True)
        return 0

    lax.fori_loop(0, NCHUNK, body, 0)
    plsc.subcore_barrier()
    pltpu.sync_copy(acc_sh.at[pl.ds(base, 400)],
                    out_hbm.at[c].at[pl.ds(base, 400)])

    @pl.when(s < NS - 1)
    def _():
        pltpu.sync_copy(acc_sh.at[pl.ds(base + 400, 240)],
                        out_hbm.at[c].at[pl.ds(base + 400, 240)])


def _tc_quant_body(x_ref, ba_ref, w_ref, degt_ref, g_ref):
    x = x_ref[...]
    ba = ba_ref[...]
    w = w_ref[...]
    degt = degt_ref[...]
    deg = jnp.sum(degt, axis=1, keepdims=True) + 1.0
    dis = 1.0 / jnp.sqrt(deg)

    big = jnp.float32(1e30)
    mn_row = jnp.zeros_like(dis)
    sc_row = jnp.ones_like(dis)
    qm_row = jnp.ones_like(dis)
    for bv in (2, 4, 8):
        qmax = jnp.float32(2.0 ** bv - 1.0)
        m = ba == bv
        mn = jnp.min(jnp.where(m, x, big))
        mx = jnp.max(jnp.where(m, x, -big))
        sc = (mx - mn) / qmax
        mn_row = jnp.where(m, mn, mn_row)
        sc_row = jnp.where(m, sc, sc_row)
        qm_row = jnp.where(m, qmax, qm_row)
    xq = jnp.clip(jnp.round((x - mn_row) / sc_row), 0.0, qm_row) * sc_row + mn_row

    mnw = jnp.min(w)
    mxw = jnp.max(w)
    scw = (mxw - mnw) / 15.0
    wq = jnp.clip(jnp.round((w - mnw) / scw), 0.0, 15.0) * scw + mnw

    h = lax.dot_general(xq, wq, (((1,), (1,)), ((), ())),
                        preferred_element_type=jnp.float32)
    g_ref[...] = h * dis


def _tc_combine_body(ap_ref, g_ref, degt_ref, b_ref, out_ref):
    deg = jnp.sum(degt_ref[...], axis=1, keepdims=True) + 1.0
    dis = 1.0 / jnp.sqrt(deg)
    out_ref[...] = dis * (ap_ref[0] + ap_ref[1] + g_ref[...]) + b_ref[...]


@functools.cache
def _sc_kernels():
    mesh = plsc.VectorSubcoreMesh(core_axis_name="c", subcore_axis_name="s",
                                  num_cores=NC, num_subcores=NS)
    params = pltpu.CompilerParams(needs_layout_passes=False)
    sc_degree = functools.partial(
        pl.kernel,
        out_type=jax.ShapeDtypeStruct((NW * N,), jnp.float32),
        mesh=mesh,
        compiler_params=params,
        scratch_types=[
            pltpu.VMEM((NCHUNK, CH), jnp.int32),
            pltpu.VMEM((N,), jnp.float32),
        ],
    )(_sc_degree_body)
    sc_scatter = functools.partial(
        pl.kernel,
        out_type=jax.ShapeDtypeStruct((NC, N, D), jnp.float32),
        mesh=mesh,
        compiler_params=params,
        scratch_types=[
            pltpu.VMEM((NCHUNK, CH), jnp.int32),
            pltpu.VMEM((NCHUNK, CH), jnp.int32),
            pltpu.VMEM((CH, D), jnp.float32),
            pltpu.VMEM_SHARED((N, D), jnp.float32),
            pltpu.SemaphoreType.DMA,
        ],
    )(_sc_scatter_body)
    return sc_degree, sc_scatter


def kernel(x, edge_index, bit_assign, W, b):
    sc_degree, sc_scatter = _sc_kernels()
    srcr = edge_index[0].reshape(NW, NCHUNK, CH)
    dstr = edge_index[1].reshape(NW, NCHUNK, CH)
    ba2d = bit_assign[:, None]

    degp = sc_degree(dstr).reshape(NW, N)
    degt = degp.T

    g = pl.pallas_call(
        _tc_quant_body,
        out_shape=jax.ShapeDtypeStruct((N, D), jnp.float32),
    )(x, ba2d, W, degt)

    ap = sc_scatter(g, srcr, dstr)

    BR = 2000
    out = pl.pallas_call(
        _tc_combine_body,
        grid=(N // BR,),
        in_specs=[
            pl.BlockSpec((NC, BR, D), lambda i: (0, i, 0)),
            pl.BlockSpec((BR, D), lambda i: (i, 0)),
            pl.BlockSpec((BR, NW), lambda i: (i, 0)),
            pl.BlockSpec((1, D), lambda i: (0, 0)),
        ],
        out_specs=pl.BlockSpec((BR, D), lambda i: (i, 0)),
        out_shape=jax.ShapeDtypeStruct((N, D), jnp.float32),
    )(ap, g, degt, b[None, :])
    return out
